# baseline (speedup 1.0000x reference)
import functools

import jax
import jax.numpy as jnp
from jax import lax
from jax.experimental import pallas as pl
from jax.experimental.pallas import tpu as pltpu
from jax.experimental.pallas import tpu_sc as plsc

D_MODEL = 4096
NUM_EXPERTS = 64
N_TOK = 32768
K_MAX = 0.3
LOGIT_STD_EMA = 1.0
LOGIT_STD_ALPHA = 0.99

T1 = 1024
T2 = 4096

N_SUBCORES = 32
TOK_PER_SUB = N_TOK // N_SUBCORES
GROUPS = TOK_PER_SUB // 16


def _pass1(x_ref, wt_ref, beta_col_ref, beta_row_ref, z_ref, zt_ref,
           b16_ref, brow_ref, acc_ref):
    z = jnp.dot(x_ref[...], wt_ref[...], preferred_element_type=jnp.float32)
    z_ref[...] = z
    zt_ref[...] = z.T.reshape(1, NUM_EXPERTS, T1)
    i = pl.program_id(0)

    @pl.when(i == 0)
    def _():
        acc_ref[0] = 0.0
        acc_ref[1] = 0.0

    acc_ref[0] += jnp.sum(z)
    acc_ref[1] += jnp.sum(z * z)

    @pl.when(i == pl.num_programs(0) - 1)
    def _():
        n = float(N_TOK * NUM_EXPERTS)
        s = acc_ref[0]
        ss = acc_ref[1]
        var = (ss - s * s / n) / (n - 1.0)
        ema = LOGIT_STD_ALPHA * LOGIT_STD_EMA + (1.0 - LOGIT_STD_ALPHA) * jnp.sqrt(var)
        b16_ref[...] = jnp.clip(beta_col_ref[...], -K_MAX, K_MAX) * ema
        brow_ref[...] = jnp.clip(beta_row_ref[...], -K_MAX, K_MAX) * ema


def _pass2b(brow_ref, z_ref, zb_ref):
    zb_ref[...] = z_ref[...] + brow_ref[...]


def _pass2c(p_soa_ref, i_soa_ref, p_ref, i_ref):
    p_ref[...] = p_soa_ref[...].T
    i_ref[...] = i_soa_ref[...].T


def _sc_route(zt_hbm, b16_hbm, probs_hbm, idx_hbm, zloc, bloc, p1b, p2b,
              i1b, i2b):
    wid = lax.axis_index("s") * 2 + lax.axis_index("c")
    base = wid * TOK_PER_SUB
    pltpu.sync_copy(zt_hbm.at[wid], zloc)
    pltpu.sync_copy(b16_hbm, bloc)

    def body(g, carry):
        off = g * 16
        m1 = zloc[0, pl.ds(off, 16)] + bloc[0, :]
        i1 = jnp.zeros((16,), jnp.float32)
        m2 = jnp.full((16,), -jnp.inf, jnp.float32)
        i2 = jnp.zeros((16,), jnp.float32)
        for e in range(1, NUM_EXPERTS):
            v = zloc[e, pl.ds(off, 16)] + bloc[e, :]
            ef = jnp.full((16,), float(e), jnp.float32)
            new1 = v > m1
            cand = jnp.where(new1, m1, v)
            cand_i = jnp.where(new1, i1, ef)
            new2 = cand > m2
            m2 = jnp.where(new2, cand, m2)
            i2 = jnp.where(new2, cand_i, i2)
            m1 = jnp.where(new1, v, m1)
            i1 = jnp.where(new1, ef, i1)
        r = jnp.exp(m2 - m1)
        p1 = 1.0 / (1.0 + r)
        p2 = r * p1
        p1b[pl.ds(off, 16)] = p1
        p2b[pl.ds(off, 16)] = p2
        i1b[pl.ds(off, 16)] = i1.astype(jnp.int32)
        i2b[pl.ds(off, 16)] = i2.astype(jnp.int32)
        return carry

    lax.fori_loop(0, GROUPS, body, 0)
    pltpu.sync_copy(p1b, probs_hbm.at[0, pl.ds(base, TOK_PER_SUB)])
    pltpu.sync_copy(p2b, probs_hbm.at[1, pl.ds(base, TOK_PER_SUB)])
    pltpu.sync_copy(i1b, idx_hbm.at[0, pl.ds(base, TOK_PER_SUB)])
    pltpu.sync_copy(i2b, idx_hbm.at[1, pl.ds(base, TOK_PER_SUB)])


@functools.partial(jax.jit, static_argnames=())
def kernel(hidden_states, W_gate, beta_coeff, top_k):
    del top_k
    wt = W_gate.T
    beta_col = jnp.broadcast_to(beta_coeff.reshape(NUM_EXPERTS, 1),
                                (NUM_EXPERTS, 16))
    beta_row = beta_coeff.reshape(1, NUM_EXPERTS)

    z_clean, z_t3, beta16, brow = pl.pallas_call(
        _pass1,
        grid=(N_TOK // T1,),
        in_specs=[
            pl.BlockSpec((T1, D_MODEL), lambda i: (i, 0)),
            pl.BlockSpec((D_MODEL, NUM_EXPERTS), lambda i: (0, 0)),
            pl.BlockSpec((NUM_EXPERTS, 16), lambda i: (0, 0)),
            pl.BlockSpec((1, NUM_EXPERTS), lambda i: (0, 0)),
        ],
        out_specs=[
            pl.BlockSpec((T1, NUM_EXPERTS), lambda i: (i, 0)),
            pl.BlockSpec((1, NUM_EXPERTS, T1), lambda i: (i, 0, 0)),
            pl.BlockSpec((NUM_EXPERTS, 16), lambda i: (0, 0)),
            pl.BlockSpec((1, NUM_EXPERTS), lambda i: (0, 0)),
        ],
        out_shape=[
            jax.ShapeDtypeStruct((N_TOK, NUM_EXPERTS), jnp.float32),
            jax.ShapeDtypeStruct((N_SUBCORES, NUM_EXPERTS, TOK_PER_SUB),
                                 jnp.float32),
            jax.ShapeDtypeStruct((NUM_EXPERTS, 16), jnp.float32),
            jax.ShapeDtypeStruct((1, NUM_EXPERTS), jnp.float32),
        ],
        scratch_shapes=[pltpu.SMEM((2,), jnp.float32)],
    )(hidden_states, wt, beta_col, beta_row)

    sc_fn = pl.kernel(
        _sc_route,
        out_type=[
            jax.ShapeDtypeStruct((2, N_TOK), jnp.float32),
            jax.ShapeDtypeStruct((2, N_TOK), jnp.int32),
        ],
        mesh=plsc.VectorSubcoreMesh(core_axis_name="c", subcore_axis_name="s"),
        scratch_types=[
            pltpu.VMEM((NUM_EXPERTS, TOK_PER_SUB), jnp.float32),
            pltpu.VMEM((NUM_EXPERTS, 16), jnp.float32),
            pltpu.VMEM((TOK_PER_SUB,), jnp.float32),
            pltpu.VMEM((TOK_PER_SUB,), jnp.float32),
            pltpu.VMEM((TOK_PER_SUB,), jnp.int32),
            pltpu.VMEM((TOK_PER_SUB,), jnp.int32),
        ],
    )
    probs_soa, idx_soa = sc_fn(z_t3, beta16)

    probs, idx = pl.pallas_call(
        _pass2c,
        grid=(N_TOK // T2,),
        in_specs=[
            pl.BlockSpec((2, T2), lambda i: (0, i)),
            pl.BlockSpec((2, T2), lambda i: (0, i)),
        ],
        out_specs=[
            pl.BlockSpec((T2, 2), lambda i: (i, 0)),
            pl.BlockSpec((T2, 2), lambda i: (i, 0)),
        ],
        out_shape=[
            jax.ShapeDtypeStruct((N_TOK, 2), jnp.float32),
            jax.ShapeDtypeStruct((N_TOK, 2), jnp.int32),
        ],
    )(probs_soa, idx_soa)

    zb = pl.pallas_call(
        _pass2b,
        grid=(N_TOK // T2,),
        in_specs=[
            pl.BlockSpec((1, NUM_EXPERTS), lambda i: (0, 0)),
            pl.BlockSpec((T2, NUM_EXPERTS), lambda i: (i, 0)),
        ],
        out_specs=pl.BlockSpec((T2, NUM_EXPERTS), lambda i: (i, 0)),
        out_shape=jax.ShapeDtypeStruct((N_TOK, NUM_EXPERTS), jnp.float32),
    )(brow, z_clean)

    return probs, idx, z_clean, zb


# speedup vs baseline: 1.0077x; 1.0077x over previous
import functools

import jax
import jax.numpy as jnp
from jax import lax
from jax.experimental import pallas as pl
from jax.experimental.pallas import tpu as pltpu
from jax.experimental.pallas import tpu_sc as plsc

D_MODEL = 4096
NUM_EXPERTS = 64
N_TOK = 32768
K_MAX = 0.3
LOGIT_STD_EMA = 1.0
LOGIT_STD_ALPHA = 0.99

T1 = 1024
T2 = 4096

N_SUBCORES = 32
TOK_PER_SUB = N_TOK // N_SUBCORES
GROUPS = TOK_PER_SUB // 16


def _pass1(x_ref, wt_ref, beta_col_ref, beta_row_ref, z_ref, zt_ref,
           b16_ref, brow_ref, acc_ref):
    z = jnp.dot(x_ref[...], wt_ref[...], preferred_element_type=jnp.float32)
    z_ref[...] = z
    zt_ref[...] = z.T.reshape(1, NUM_EXPERTS, T1)
    i = pl.program_id(0)

    @pl.when(i == 0)
    def _():
        acc_ref[0] = 0.0
        acc_ref[1] = 0.0

    acc_ref[0] += jnp.sum(z)
    acc_ref[1] += jnp.sum(z * z)

    @pl.when(i == pl.num_programs(0) - 1)
    def _():
        n = float(N_TOK * NUM_EXPERTS)
        s = acc_ref[0]
        ss = acc_ref[1]
        var = (ss - s * s / n) / (n - 1.0)
        ema = LOGIT_STD_ALPHA * LOGIT_STD_EMA + (1.0 - LOGIT_STD_ALPHA) * jnp.sqrt(var)
        b16_ref[...] = jnp.clip(beta_col_ref[...], -K_MAX, K_MAX) * ema
        brow_ref[...] = jnp.clip(beta_row_ref[...], -K_MAX, K_MAX) * ema


def _pass2b(brow_ref, z_ref, zb_ref):
    zb_ref[...] = z_ref[...] + brow_ref[...]


def _pass2c(p_soa_ref, i_soa_ref, p_ref, i_ref):
    p_ref[...] = p_soa_ref[...].T
    i_ref[...] = i_soa_ref[...].T


def _sc_route(zt_hbm, b16_hbm, probs_hbm, idx_hbm, zloc, bloc, p1b, p2b,
              i1b, i2b):
    wid = lax.axis_index("s") * 2 + lax.axis_index("c")
    base = wid * TOK_PER_SUB
    pltpu.sync_copy(zt_hbm.at[wid], zloc)
    pltpu.sync_copy(b16_hbm, bloc)

    def body(g, carry):

        def chain(e0):
            m1 = zloc[e0, pl.ds(off, 16)] + bloc[e0, :]
            i1 = jnp.full((16,), float(e0), jnp.float32)
            m2 = jnp.full((16,), -jnp.inf, jnp.float32)
            i2 = jnp.zeros((16,), jnp.float32)
            for e in range(e0 + 1, e0 + 16):
                v = zloc[e, pl.ds(off, 16)] + bloc[e, :]
                ef = jnp.full((16,), float(e), jnp.float32)
                new1 = v > m1
                cand = jnp.where(new1, m1, v)
                cand_i = jnp.where(new1, i1, ef)
                new2 = cand > m2
                m2 = jnp.where(new2, cand, m2)
                i2 = jnp.where(new2, cand_i, i2)
                m1 = jnp.where(new1, v, m1)
                i1 = jnp.where(new1, ef, i1)
            return m1, i1, m2, i2

        def merge(a, b):
            am1, ai1, am2, ai2 = a
            bm1, bi1, bm2, bi2 = b
            a_first = am1 >= bm1
            m1 = jnp.where(a_first, am1, bm1)
            i1 = jnp.where(a_first, ai1, bi1)
            a2_ge = am2 >= bm1
            m2a = jnp.where(a2_ge, am2, bm1)
            i2a = jnp.where(a2_ge, ai2, bi1)
            b2_gt = bm2 > am1
            m2b = jnp.where(b2_gt, bm2, am1)
            i2b = jnp.where(b2_gt, bi2, ai1)
            m2 = jnp.where(a_first, m2a, m2b)
            i2 = jnp.where(a_first, i2a, i2b)
            return m1, i1, m2, i2

        for half in range(2):
            off = g * 32 + half * 16
            m1, i1, m2, i2 = merge(merge(chain(0), chain(16)),
                                   merge(chain(32), chain(48)))
            r = jnp.exp(m2 - m1)
            p1 = 1.0 / (1.0 + r)
            p2 = r * p1
            p1b[pl.ds(off, 16)] = p1
            p2b[pl.ds(off, 16)] = p2
            i1b[pl.ds(off, 16)] = i1.astype(jnp.int32)
            i2b[pl.ds(off, 16)] = i2.astype(jnp.int32)
        return carry

    lax.fori_loop(0, GROUPS // 2, body, 0)
    pltpu.sync_copy(p1b, probs_hbm.at[0, pl.ds(base, TOK_PER_SUB)])
    pltpu.sync_copy(p2b, probs_hbm.at[1, pl.ds(base, TOK_PER_SUB)])
    pltpu.sync_copy(i1b, idx_hbm.at[0, pl.ds(base, TOK_PER_SUB)])
    pltpu.sync_copy(i2b, idx_hbm.at[1, pl.ds(base, TOK_PER_SUB)])


@functools.partial(jax.jit, static_argnames=())
def kernel(hidden_states, W_gate, beta_coeff, top_k):
    del top_k
    wt = W_gate.T
    beta_col = jnp.broadcast_to(beta_coeff.reshape(NUM_EXPERTS, 1),
                                (NUM_EXPERTS, 16))
    beta_row = beta_coeff.reshape(1, NUM_EXPERTS)

    z_clean, z_t3, beta16, brow = pl.pallas_call(
        _pass1,
        grid=(N_TOK // T1,),
        in_specs=[
            pl.BlockSpec((T1, D_MODEL), lambda i: (i, 0)),
            pl.BlockSpec((D_MODEL, NUM_EXPERTS), lambda i: (0, 0)),
            pl.BlockSpec((NUM_EXPERTS, 16), lambda i: (0, 0)),
            pl.BlockSpec((1, NUM_EXPERTS), lambda i: (0, 0)),
        ],
        out_specs=[
            pl.BlockSpec((T1, NUM_EXPERTS), lambda i: (i, 0)),
            pl.BlockSpec((1, NUM_EXPERTS, T1), lambda i: (i, 0, 0)),
            pl.BlockSpec((NUM_EXPERTS, 16), lambda i: (0, 0)),
            pl.BlockSpec((1, NUM_EXPERTS), lambda i: (0, 0)),
        ],
        out_shape=[
            jax.ShapeDtypeStruct((N_TOK, NUM_EXPERTS), jnp.float32),
            jax.ShapeDtypeStruct((N_SUBCORES, NUM_EXPERTS, TOK_PER_SUB),
                                 jnp.float32),
            jax.ShapeDtypeStruct((NUM_EXPERTS, 16), jnp.float32),
            jax.ShapeDtypeStruct((1, NUM_EXPERTS), jnp.float32),
        ],
        scratch_shapes=[pltpu.SMEM((2,), jnp.float32)],
    )(hidden_states, wt, beta_col, beta_row)

    sc_fn = pl.kernel(
        _sc_route,
        out_type=[
            jax.ShapeDtypeStruct((2, N_TOK), jnp.float32),
            jax.ShapeDtypeStruct((2, N_TOK), jnp.int32),
        ],
        mesh=plsc.VectorSubcoreMesh(core_axis_name="c", subcore_axis_name="s"),
        scratch_types=[
            pltpu.VMEM((NUM_EXPERTS, TOK_PER_SUB), jnp.float32),
            pltpu.VMEM((NUM_EXPERTS, 16), jnp.float32),
            pltpu.VMEM((TOK_PER_SUB,), jnp.float32),
            pltpu.VMEM((TOK_PER_SUB,), jnp.float32),
            pltpu.VMEM((TOK_PER_SUB,), jnp.int32),
            pltpu.VMEM((TOK_PER_SUB,), jnp.int32),
        ],
    )
    probs_soa, idx_soa = sc_fn(z_t3, beta16)

    probs, idx = pl.pallas_call(
        _pass2c,
        grid=(N_TOK // T2,),
        in_specs=[
            pl.BlockSpec((2, T2), lambda i: (0, i)),
            pl.BlockSpec((2, T2), lambda i: (0, i)),
        ],
        out_specs=[
            pl.BlockSpec((T2, 2), lambda i: (i, 0)),
            pl.BlockSpec((T2, 2), lambda i: (i, 0)),
        ],
        out_shape=[
            jax.ShapeDtypeStruct((N_TOK, 2), jnp.float32),
            jax.ShapeDtypeStruct((N_TOK, 2), jnp.int32),
        ],
    )(probs_soa, idx_soa)

    zb = pl.pallas_call(
        _pass2b,
        grid=(N_TOK // T2,),
        in_specs=[
            pl.BlockSpec((1, NUM_EXPERTS), lambda i: (0, 0)),
            pl.BlockSpec((T2, NUM_EXPERTS), lambda i: (i, 0)),
        ],
        out_specs=pl.BlockSpec((T2, NUM_EXPERTS), lambda i: (i, 0)),
        out_shape=jax.ShapeDtypeStruct((N_TOK, NUM_EXPERTS), jnp.float32),
    )(brow, z_clean)

    return probs, idx, z_clean, zb
